# Initial kernel scaffold; baseline (speedup 1.0000x reference)
import jax
import jax.numpy as jnp
from jax.experimental import pallas as pl


def kernel(x, edge_index, W1, b1, W2, b2, gamma, beta, epsilon):
    raise NotImplementedError("write your pallas kernel here")


# speedup vs baseline: 6.7395x; 6.7395x over previous
import functools

import jax
import jax.numpy as jnp
from jax import lax
from jax.experimental import pallas as pl
from jax.experimental.pallas import tpu as pltpu
from jax.experimental.pallas import tpu_sc as plsc

N = 10000
D = 128
E = 320000
BN_EPS_CONST = 1e-5

NC = 2
NS = 16
NW = NC * NS
CK = 80
CHUNKS_PER_W = E // NW // CK

ROW_STRIDE = 624
ZCHUNK = 80
NZ = 8


def _sc_scatter_body(x_hbm, src_hbm, dst_hbm, out_hbm, acc, src_v, dst_v, rows_v, sem):
    cid = lax.axis_index("c")
    sid = lax.axis_index("s")
    wid = sid * NC + cid
    row0 = sid * ROW_STRIDE

    def zbody(i, _):
        r = i // (D // 16)
        c = (i % (D // 16)) * 16
        rows_v[r, pl.ds(c, 16)] = jnp.zeros((16,), jnp.float32)
        return 0
    lax.fori_loop(0, ZCHUNK * (D // 16), zbody, 0)
    def zcopy(i, _):
        pltpu.sync_copy(rows_v, acc.at[pl.ds(row0 + i * ZCHUNK, ZCHUNK)])
        return 0
    lax.fori_loop(0, NZ, zcopy, 0)
    plsc.subcore_barrier()

    pltpu.sync_copy(src_hbm.at[wid], src_v)
    pltpu.sync_copy(dst_hbm.at[wid], dst_v)

    def edge_body(j, _):
        pltpu.async_copy(x_hbm.at[src_v.at[j]], rows_v, sem).wait()
        pltpu.sync_copy(rows_v, acc.at[dst_v.at[j]], add=True)
        return 0
    lax.fori_loop(0, CHUNKS_PER_W, edge_body, 0)

    plsc.subcore_barrier()

    def ocopy(i, _):
        pltpu.sync_copy(acc.at[pl.ds(row0 + i * ZCHUNK, ZCHUNK)],
                        out_hbm.at[cid, pl.ds(row0 + i * ZCHUNK, ZCHUNK)])
        return 0
    lax.fori_loop(0, NZ, ocopy, 0)


_sc_scatter = functools.partial(
    pl.kernel,
    out_type=jax.ShapeDtypeStruct((NC, N, D), jnp.float32),
    mesh=plsc.VectorSubcoreMesh(core_axis_name="c", subcore_axis_name="s"),
    scratch_types=[
        pltpu.VMEM_SHARED((N, D), jnp.float32),
        pltpu.VMEM((CHUNKS_PER_W, CK), jnp.int32),
        pltpu.VMEM((CHUNKS_PER_W, CK), jnp.int32),
        pltpu.VMEM((CK, D), jnp.float32),
        pltpu.SemaphoreType.DMA,
    ],
)(_sc_scatter_body)


BR = 1000


def _mlp_body(p_ref, x_ref, epsb_ref, W1_ref, b1_ref, W2_ref, b2_ref,
              h_ref, stats_ref, acc_ref):
    i = pl.program_id(0)

    @pl.when(i == 0)
    def _():
        acc_ref[...] = jnp.zeros_like(acc_ref)

    agg = p_ref[0] + p_ref[1] + epsb_ref[0, 0] * x_ref[...]
    h1 = jax.lax.dot_general(agg, W1_ref[...], (((1,), (1,)), ((), ())),
                             preferred_element_type=jnp.float32)
    h1 = jnp.maximum(h1 + b1_ref[...], 0.0)
    h2 = jax.lax.dot_general(h1, W2_ref[...], (((1,), (1,)), ((), ())),
                             preferred_element_type=jnp.float32)
    h2 = h2 + b2_ref[...]
    h_ref[...] = h2
    acc_ref[0:1, :] += jnp.sum(h2, axis=0, keepdims=True)
    acc_ref[1:2, :] += jnp.sum(h2 * h2, axis=0, keepdims=True)
    stats_ref[...] = acc_ref[...]


def _bn_body(h_ref, stats_ref, gamma_ref, beta_ref, o_ref):
    mean = stats_ref[0:1, :] * (1.0 / N)
    var = stats_ref[1:2, :] * (1.0 / N) - mean * mean
    inv = jax.lax.rsqrt(var + BN_EPS_CONST)
    o_ref[...] = jnp.maximum(
        (h_ref[...] - mean) * (inv * gamma_ref[...]) + beta_ref[...], 0.0)


def kernel(x, edge_index, W1, b1, W2, b2, gamma, beta, epsilon):
    src = edge_index[0].reshape(NW, CHUNKS_PER_W, CK)
    dst = edge_index[1].reshape(NW, CHUNKS_PER_W, CK)
    partials = _sc_scatter(x, src, dst)

    epsb = jnp.reshape(1.0 + epsilon, (1, 1)).astype(jnp.float32)
    nb = N // BR
    h, stats = pl.pallas_call(
        _mlp_body,
        grid=(nb,),
        in_specs=[
            pl.BlockSpec((NC, BR, D), lambda i: (0, i, 0)),
            pl.BlockSpec((BR, D), lambda i: (i, 0)),
            pl.BlockSpec((1, 1), lambda i: (0, 0)),
            pl.BlockSpec((D, D), lambda i: (0, 0)),
            pl.BlockSpec((1, D), lambda i: (0, 0)),
            pl.BlockSpec((D, D), lambda i: (0, 0)),
            pl.BlockSpec((1, D), lambda i: (0, 0)),
        ],
        out_specs=[
            pl.BlockSpec((BR, D), lambda i: (i, 0)),
            pl.BlockSpec((2, D), lambda i: (0, 0)),
        ],
        out_shape=[
            jax.ShapeDtypeStruct((N, D), jnp.float32),
            jax.ShapeDtypeStruct((2, D), jnp.float32),
        ],
        scratch_shapes=[pltpu.VMEM((2, D), jnp.float32)],
    )(partials, x, epsb, W1, b1.reshape(1, D), W2, b2.reshape(1, D))

    out = pl.pallas_call(
        _bn_body,
        grid=(nb,),
        in_specs=[
            pl.BlockSpec((BR, D), lambda i: (i, 0)),
            pl.BlockSpec((2, D), lambda i: (0, 0)),
            pl.BlockSpec((1, D), lambda i: (0, 0)),
            pl.BlockSpec((1, D), lambda i: (0, 0)),
        ],
        out_specs=pl.BlockSpec((BR, D), lambda i: (i, 0)),
        out_shape=jax.ShapeDtypeStruct((N, D), jnp.float32),
    )(h, stats, gamma.reshape(1, D), beta.reshape(1, D))
    return out


# speedup vs baseline: 9.6945x; 1.4385x over previous
import functools

import jax
import jax.numpy as jnp
from jax import lax
from jax.experimental import pallas as pl
from jax.experimental.pallas import tpu as pltpu
from jax.experimental.pallas import tpu_sc as plsc

N = 10000
D = 128
E = 320000
BN_EPS_CONST = 1e-5

NC = 2
NS = 16
NW = NC * NS
CK = 80
CHUNKS_PER_W = E // NW // CK
G = 25
NG = CHUNKS_PER_W // G

ROW_STRIDE = 624
ZCHUNK = 80
NZ = 8


def _sc_scatter_body(x_hbm, src_hbm, dst_hbm, out_hbm, acc, src_v, dst_v,
                     rows_v, rows_w, sem0, sem1):
    cid = lax.axis_index("c")
    sid = lax.axis_index("s")
    wid = sid * NC + cid
    row0 = sid * ROW_STRIDE

    def zbody(i, _):
        r = i // (D // 16)
        c = (i % (D // 16)) * 16
        rows_v[r, pl.ds(c, 16)] = jnp.zeros((16,), jnp.float32)
        return 0
    lax.fori_loop(0, ZCHUNK * (D // 16), zbody, 0)
    def zcopy(i, _):
        pltpu.sync_copy(rows_v, acc.at[pl.ds(row0 + i * ZCHUNK, ZCHUNK)])
        return 0
    lax.fori_loop(0, NZ, zcopy, 0)
    plsc.subcore_barrier()

    bufs = (rows_v, rows_w)
    sems = (sem0, sem1)

    def start_gather(j, b):
        pltpu.async_copy(x_hbm.at[src_v.at[j]], bufs[b], sems[b])

    def wait_gather(b):
        pltpu.make_async_copy(x_hbm.at[src_v.at[0]], bufs[b], sems[b]).wait()

    def scatter(j, b):
        pltpu.sync_copy(bufs[b], acc.at[dst_v.at[j]], add=True)

    def group_body(g, _):
        pltpu.sync_copy(src_hbm.at[wid, g], src_v)
        pltpu.sync_copy(dst_hbm.at[wid, g], dst_v)
        start_gather(0, 0)

        def edge_body(t, _):
            j = 2 * t
            start_gather(j + 1, 1)
            wait_gather(0)
            scatter(j, 0)
            start_gather(j + 2, 0)
            wait_gather(1)
            scatter(j + 1, 1)
            return 0
        lax.fori_loop(0, (G - 1) // 2, edge_body, 0, unroll=False)
        wait_gather(0)
        scatter(G - 1, 0)
        return 0
    lax.fori_loop(0, NG, group_body, 0, unroll=False)

    plsc.subcore_barrier()

    def ocopy(i, _):
        pltpu.sync_copy(acc.at[pl.ds(row0 + i * ZCHUNK, ZCHUNK)],
                        out_hbm.at[cid, pl.ds(row0 + i * ZCHUNK, ZCHUNK)])
        return 0
    lax.fori_loop(0, NZ, ocopy, 0)


_sc_scatter = functools.partial(
    pl.kernel,
    out_type=jax.ShapeDtypeStruct((NC, N, D), jnp.float32),
    mesh=plsc.VectorSubcoreMesh(core_axis_name="c", subcore_axis_name="s"),
    scratch_types=[
        pltpu.VMEM_SHARED((N, D), jnp.float32),
        pltpu.VMEM((G, CK), jnp.int32),
        pltpu.VMEM((G, CK), jnp.int32),
        pltpu.VMEM((CK, D), jnp.float32),
        pltpu.VMEM((CK, D), jnp.float32),
        pltpu.SemaphoreType.DMA,
        pltpu.SemaphoreType.DMA,
    ],
)(_sc_scatter_body)


BR = 1000


def _mlp_body(p_ref, x_ref, epsb_ref, W1_ref, b1_ref, W2_ref, b2_ref,
              h_ref, stats_ref, acc_ref):
    i = pl.program_id(0)

    @pl.when(i == 0)
    def _():
        acc_ref[...] = jnp.zeros_like(acc_ref)

    agg = p_ref[0] + p_ref[1] + epsb_ref[0, 0] * x_ref[...]
    h1 = jax.lax.dot_general(agg, W1_ref[...], (((1,), (1,)), ((), ())),
                             preferred_element_type=jnp.float32)
    h1 = jnp.maximum(h1 + b1_ref[...], 0.0)
    h2 = jax.lax.dot_general(h1, W2_ref[...], (((1,), (1,)), ((), ())),
                             preferred_element_type=jnp.float32)
    h2 = h2 + b2_ref[...]
    h_ref[...] = h2
    acc_ref[0:1, :] += jnp.sum(h2, axis=0, keepdims=True)
    acc_ref[1:2, :] += jnp.sum(h2 * h2, axis=0, keepdims=True)
    stats_ref[...] = acc_ref[...]


def _bn_body(h_ref, stats_ref, gamma_ref, beta_ref, o_ref):
    mean = stats_ref[0:1, :] * (1.0 / N)
    var = stats_ref[1:2, :] * (1.0 / N) - mean * mean
    inv = jax.lax.rsqrt(var + BN_EPS_CONST)
    o_ref[...] = jnp.maximum(
        (h_ref[...] - mean) * (inv * gamma_ref[...]) + beta_ref[...], 0.0)


def kernel(x, edge_index, W1, b1, W2, b2, gamma, beta, epsilon):
    src = edge_index[0].reshape(NW, NG, G, CK)
    dst = edge_index[1].reshape(NW, NG, G, CK)
    partials = _sc_scatter(x, src, dst)

    epsb = jnp.reshape(1.0 + epsilon, (1, 1)).astype(jnp.float32)
    nb = N // BR
    h, stats = pl.pallas_call(
        _mlp_body,
        grid=(nb,),
        in_specs=[
            pl.BlockSpec((NC, BR, D), lambda i: (0, i, 0)),
            pl.BlockSpec((BR, D), lambda i: (i, 0)),
            pl.BlockSpec((1, 1), lambda i: (0, 0)),
            pl.BlockSpec((D, D), lambda i: (0, 0)),
            pl.BlockSpec((1, D), lambda i: (0, 0)),
            pl.BlockSpec((D, D), lambda i: (0, 0)),
            pl.BlockSpec((1, D), lambda i: (0, 0)),
        ],
        out_specs=[
            pl.BlockSpec((BR, D), lambda i: (i, 0)),
            pl.BlockSpec((2, D), lambda i: (0, 0)),
        ],
        out_shape=[
            jax.ShapeDtypeStruct((N, D), jnp.float32),
            jax.ShapeDtypeStruct((2, D), jnp.float32),
        ],
        scratch_shapes=[pltpu.VMEM((2, D), jnp.float32)],
    )(partials, x, epsb, W1, b1.reshape(1, D), W2, b2.reshape(1, D))

    out = pl.pallas_call(
        _bn_body,
        grid=(nb,),
        in_specs=[
            pl.BlockSpec((BR, D), lambda i: (i, 0)),
            pl.BlockSpec((2, D), lambda i: (0, 0)),
            pl.BlockSpec((1, D), lambda i: (0, 0)),
            pl.BlockSpec((1, D), lambda i: (0, 0)),
        ],
        out_specs=pl.BlockSpec((BR, D), lambda i: (i, 0)),
        out_shape=jax.ShapeDtypeStruct((N, D), jnp.float32),
    )(h, stats, gamma.reshape(1, D), beta.reshape(1, D))
    return out


# speedup vs baseline: 10.7238x; 1.1062x over previous
import functools

import jax
import jax.numpy as jnp
from jax import lax
from jax.experimental import pallas as pl
from jax.experimental.pallas import tpu as pltpu
from jax.experimental.pallas import tpu_sc as plsc

N = 10000
D = 128
E = 320000
BN_EPS_CONST = 1e-5

NC = 2
NS = 16
NW = NC * NS
CK = 80
CHUNKS_PER_W = E // NW // CK
G = 25
NG = CHUNKS_PER_W // G

ROW_STRIDE = 624
ZCHUNK = 80
NZ = 8


def _sc_scatter_body(x_hbm, src_hbm, dst_hbm, out_hbm, acc, src_v, dst_v,
                     rows_0, rows_1, rows_2, gsem0, gsem1, gsem2,
                     ssem0, ssem1, ssem2):
    cid = lax.axis_index("c")
    sid = lax.axis_index("s")
    wid = sid * NC + cid
    row0 = sid * ROW_STRIDE

    def zbody(i, _):
        r = i // (D // 16)
        c = (i % (D // 16)) * 16
        rows_0[r, pl.ds(c, 16)] = jnp.zeros((16,), jnp.float32)
        return 0
    lax.fori_loop(0, ZCHUNK * (D // 16), zbody, 0)
    def zcopy(i, _):
        pltpu.sync_copy(rows_0, acc.at[pl.ds(row0 + i * ZCHUNK, ZCHUNK)])
        return 0
    lax.fori_loop(0, NZ, zcopy, 0)
    plsc.subcore_barrier()

    bufs = (rows_0, rows_1, rows_2)
    gsems = (gsem0, gsem1, gsem2)
    ssems = (ssem0, ssem1, ssem2)

    def start_gather(j, b):
        pltpu.async_copy(x_hbm.at[src_v.at[j]], bufs[b], gsems[b])

    def wait_gather(b):
        pltpu.make_async_copy(x_hbm.at[src_v.at[0]], bufs[b], gsems[b]).wait()

    def start_scatter(j, b):
        pltpu.async_copy(bufs[b], acc.at[dst_v.at[j]], ssems[b], add=True)

    def wait_scatter(b):
        pltpu.make_async_copy(bufs[b], acc.at[dst_v.at[0]], ssems[b]).wait()

    def group_body(g, _):
        pltpu.sync_copy(src_hbm.at[wid, g], src_v)
        pltpu.sync_copy(dst_hbm.at[wid, g], dst_v)
        start_gather(0, 0)
        start_gather(1, 1)
        wait_gather(0)
        start_scatter(0, 0)
        start_gather(2, 2)

        def edge_body(t, _):
            for i in range(3):
                j = 3 * t + 1 + i
                s = (1 + i) % 3
                wait_gather(s)
                start_scatter(j, s)
                wait_scatter((s + 2) % 3)

                @pl.when(j + 2 <= G - 1)
                def _():
                    start_gather(j + 2, (s + 2) % 3)
            return 0
        lax.fori_loop(0, (G - 1) // 3, edge_body, 0, unroll=False)
        wait_scatter((G - 1) % 3)
        return 0
    lax.fori_loop(0, NG, group_body, 0, unroll=False)

    plsc.subcore_barrier()

    def ocopy(i, _):
        pltpu.sync_copy(acc.at[pl.ds(row0 + i * ZCHUNK, ZCHUNK)],
                        out_hbm.at[cid, pl.ds(row0 + i * ZCHUNK, ZCHUNK)])
        return 0
    lax.fori_loop(0, NZ, ocopy, 0)


_sc_scatter = functools.partial(
    pl.kernel,
    out_type=jax.ShapeDtypeStruct((NC, N, D), jnp.float32),
    mesh=plsc.VectorSubcoreMesh(core_axis_name="c", subcore_axis_name="s"),
    scratch_types=[
        pltpu.VMEM_SHARED((N, D), jnp.float32),
        pltpu.VMEM((G, CK), jnp.int32),
        pltpu.VMEM((G, CK), jnp.int32),
        pltpu.VMEM((CK, D), jnp.float32),
        pltpu.VMEM((CK, D), jnp.float32),
        pltpu.VMEM((CK, D), jnp.float32),
        pltpu.SemaphoreType.DMA,
        pltpu.SemaphoreType.DMA,
        pltpu.SemaphoreType.DMA,
        pltpu.SemaphoreType.DMA,
        pltpu.SemaphoreType.DMA,
        pltpu.SemaphoreType.DMA,
    ],
)(_sc_scatter_body)


BR = 1000


def _mlp_body(p_ref, x_ref, epsb_ref, W1_ref, b1_ref, W2_ref, b2_ref,
              h_ref, stats_ref, acc_ref):
    i = pl.program_id(0)

    @pl.when(i == 0)
    def _():
        acc_ref[...] = jnp.zeros_like(acc_ref)

    agg = p_ref[0] + p_ref[1] + epsb_ref[0, 0] * x_ref[...]
    h1 = jax.lax.dot_general(agg, W1_ref[...], (((1,), (1,)), ((), ())),
                             preferred_element_type=jnp.float32)
    h1 = jnp.maximum(h1 + b1_ref[...], 0.0)
    h2 = jax.lax.dot_general(h1, W2_ref[...], (((1,), (1,)), ((), ())),
                             preferred_element_type=jnp.float32)
    h2 = h2 + b2_ref[...]
    h_ref[...] = h2
    acc_ref[0:1, :] += jnp.sum(h2, axis=0, keepdims=True)
    acc_ref[1:2, :] += jnp.sum(h2 * h2, axis=0, keepdims=True)
    stats_ref[...] = acc_ref[...]


def _bn_body(h_ref, stats_ref, gamma_ref, beta_ref, o_ref):
    mean = stats_ref[0:1, :] * (1.0 / N)
    var = stats_ref[1:2, :] * (1.0 / N) - mean * mean
    inv = jax.lax.rsqrt(var + BN_EPS_CONST)
    o_ref[...] = jnp.maximum(
        (h_ref[...] - mean) * (inv * gamma_ref[...]) + beta_ref[...], 0.0)


def kernel(x, edge_index, W1, b1, W2, b2, gamma, beta, epsilon):
    src = edge_index[0].reshape(NW, NG, G, CK)
    dst = edge_index[1].reshape(NW, NG, G, CK)
    partials = _sc_scatter(x, src, dst)

    epsb = jnp.reshape(1.0 + epsilon, (1, 1)).astype(jnp.float32)
    nb = N // BR
    h, stats = pl.pallas_call(
        _mlp_body,
        grid=(nb,),
        in_specs=[
            pl.BlockSpec((NC, BR, D), lambda i: (0, i, 0)),
            pl.BlockSpec((BR, D), lambda i: (i, 0)),
            pl.BlockSpec((1, 1), lambda i: (0, 0)),
            pl.BlockSpec((D, D), lambda i: (0, 0)),
            pl.BlockSpec((1, D), lambda i: (0, 0)),
            pl.BlockSpec((D, D), lambda i: (0, 0)),
            pl.BlockSpec((1, D), lambda i: (0, 0)),
        ],
        out_specs=[
            pl.BlockSpec((BR, D), lambda i: (i, 0)),
            pl.BlockSpec((2, D), lambda i: (0, 0)),
        ],
        out_shape=[
            jax.ShapeDtypeStruct((N, D), jnp.float32),
            jax.ShapeDtypeStruct((2, D), jnp.float32),
        ],
        scratch_shapes=[pltpu.VMEM((2, D), jnp.float32)],
    )(partials, x, epsb, W1, b1.reshape(1, D), W2, b2.reshape(1, D))

    out = pl.pallas_call(
        _bn_body,
        grid=(nb,),
        in_specs=[
            pl.BlockSpec((BR, D), lambda i: (i, 0)),
            pl.BlockSpec((2, D), lambda i: (0, 0)),
            pl.BlockSpec((1, D), lambda i: (0, 0)),
            pl.BlockSpec((1, D), lambda i: (0, 0)),
        ],
        out_specs=pl.BlockSpec((BR, D), lambda i: (i, 0)),
        out_shape=jax.ShapeDtypeStruct((N, D), jnp.float32),
    )(h, stats, gamma.reshape(1, D), beta.reshape(1, D))
    return out


# speedup vs baseline: 11.4724x; 1.0698x over previous
import functools

import jax
import jax.numpy as jnp
from jax import lax
from jax.experimental import pallas as pl
from jax.experimental.pallas import tpu as pltpu
from jax.experimental.pallas import tpu_sc as plsc

N = 10000
D = 128
E = 320000
BN_EPS_CONST = 1e-5

NC = 2
NS = 16
NW = NC * NS
CK = 80
CHUNKS_PER_W = E // NW // CK
G = 25
NG = CHUNKS_PER_W // G

ROW_STRIDE = 624
ZCHUNK = 80
NZ = 8


def _sc_scatter_body(x_hbm, src_hbm, dst_hbm, out_hbm, acc, src_v, dst_v,
                     rows_0, rows_1, rows_2, gsem0, gsem1, gsem2,
                     ssem0, ssem1, ssem2):
    cid = lax.axis_index("c")
    sid = lax.axis_index("s")
    wid = sid * NC + cid
    row0 = sid * ROW_STRIDE

    def zbody(i, _):
        r = i // (D // 16)
        c = (i % (D // 16)) * 16
        rows_0[r, pl.ds(c, 16)] = jnp.zeros((16,), jnp.float32)
        return 0
    lax.fori_loop(0, ZCHUNK * (D // 16), zbody, 0)
    def zcopy(i, _):
        pltpu.sync_copy(rows_0, acc.at[pl.ds(row0 + i * ZCHUNK, ZCHUNK)])
        return 0
    lax.fori_loop(0, NZ, zcopy, 0)
    plsc.subcore_barrier()

    bufs = (rows_0, rows_1, rows_2)
    gsems = (gsem0, gsem1, gsem2)
    ssems = (ssem0, ssem1, ssem2)

    def start_gather(j, b):
        pltpu.async_copy(x_hbm.at[src_v.at[j]], bufs[b], gsems[b])

    def wait_gather(b):
        pltpu.make_async_copy(x_hbm.at[src_v.at[0]], bufs[b], gsems[b]).wait()

    def start_scatter(j, b):
        pltpu.async_copy(bufs[b], acc.at[dst_v.at[j]], ssems[b], add=True)

    def wait_scatter(b):
        pltpu.make_async_copy(bufs[b], acc.at[dst_v.at[0]], ssems[b]).wait()

    def group_body(g, _):
        pltpu.sync_copy(src_hbm.at[wid, g], src_v)
        pltpu.sync_copy(dst_hbm.at[wid, g], dst_v)
        start_gather(0, 0)
        start_gather(1, 1)
        wait_gather(0)
        start_scatter(0, 0)
        start_gather(2, 2)

        def edge_body(t, _):
            for i in range(3):
                j = 3 * t + 1 + i
                s = (1 + i) % 3
                wait_gather(s)
                start_scatter(j, s)
                wait_scatter((s + 2) % 3)

                @pl.when(j + 2 <= G - 1)
                def _():
                    start_gather(j + 2, (s + 2) % 3)
            return 0
        lax.fori_loop(0, (G - 1) // 3, edge_body, 0, unroll=False)
        wait_scatter((G - 1) % 3)
        return 0
    lax.fori_loop(0, NG, group_body, 0, unroll=False)

    plsc.subcore_barrier()

    def ocopy(i, _):
        pltpu.sync_copy(acc.at[pl.ds(row0 + i * ZCHUNK, ZCHUNK)],
                        out_hbm.at[cid, pl.ds(row0 + i * ZCHUNK, ZCHUNK)])
        return 0
    lax.fori_loop(0, NZ, ocopy, 0)


_sc_scatter = functools.partial(
    pl.kernel,
    out_type=jax.ShapeDtypeStruct((NC, N, D), jnp.float32),
    mesh=plsc.VectorSubcoreMesh(core_axis_name="c", subcore_axis_name="s"),
    scratch_types=[
        pltpu.VMEM_SHARED((N, D), jnp.float32),
        pltpu.VMEM((G, CK), jnp.int32),
        pltpu.VMEM((G, CK), jnp.int32),
        pltpu.VMEM((CK, D), jnp.float32),
        pltpu.VMEM((CK, D), jnp.float32),
        pltpu.VMEM((CK, D), jnp.float32),
        pltpu.SemaphoreType.DMA,
        pltpu.SemaphoreType.DMA,
        pltpu.SemaphoreType.DMA,
        pltpu.SemaphoreType.DMA,
        pltpu.SemaphoreType.DMA,
        pltpu.SemaphoreType.DMA,
    ],
)(_sc_scatter_body)


def _tc_body(p_ref, x_ref, epsb_ref, W1_ref, b1_ref, W2_ref, b2_ref,
             gamma_ref, beta_ref, o_ref):
    agg = p_ref[0] + p_ref[1] + epsb_ref[0, 0] * x_ref[...]
    h1 = jax.lax.dot_general(agg, W1_ref[...], (((1,), (1,)), ((), ())),
                             preferred_element_type=jnp.float32)
    h1 = jnp.maximum(h1 + b1_ref[...], 0.0)
    h2 = jax.lax.dot_general(h1, W2_ref[...], (((1,), (1,)), ((), ())),
                             preferred_element_type=jnp.float32)
    h2 = h2 + b2_ref[...]
    mean = jnp.mean(h2, axis=0, keepdims=True)
    var = jnp.mean(h2 * h2, axis=0, keepdims=True) - mean * mean
    inv = jax.lax.rsqrt(var + BN_EPS_CONST)
    o_ref[...] = jnp.maximum(
        (h2 - mean) * (inv * gamma_ref[...]) + beta_ref[...], 0.0)


def kernel(x, edge_index, W1, b1, W2, b2, gamma, beta, epsilon):
    src = edge_index[0].reshape(NW, NG, G, CK)
    dst = edge_index[1].reshape(NW, NG, G, CK)
    partials = _sc_scatter(x, src, dst)

    epsb = jnp.reshape(1.0 + epsilon, (1, 1)).astype(jnp.float32)
    out = pl.pallas_call(
        _tc_body,
        out_shape=jax.ShapeDtypeStruct((N, D), jnp.float32),
    )(partials, x, epsb, W1, b1.reshape(1, D), W2, b2.reshape(1, D),
      gamma.reshape(1, D), beta.reshape(1, D))
    return out


# speedup vs baseline: 11.5406x; 1.0059x over previous
import functools

import jax
import jax.numpy as jnp
from jax import lax
from jax.experimental import pallas as pl
from jax.experimental.pallas import tpu as pltpu
from jax.experimental.pallas import tpu_sc as plsc

N = 10000
D = 128
E = 320000
BN_EPS_CONST = 1e-5

NC = 2
NS = 16
NW = NC * NS
CK = 80
CHUNKS_PER_W = E // NW // CK
G = 25
NG = CHUNKS_PER_W // G

ROW_STRIDE = 624
ZCHUNK = 80
NZ = 8


def _sc_scatter_body(x_hbm, src_hbm, dst_hbm, out_hbm, acc, src_v, dst_v,
                     rows_0, rows_1, rows_2, gsem0, gsem1, gsem2,
                     ssem0, ssem1, ssem2):
    cid = lax.axis_index("c")
    sid = lax.axis_index("s")
    wid = sid * NC + cid
    row0 = sid * ROW_STRIDE

    def zbody(i, _):
        r = i // (D // 16)
        c = (i % (D // 16)) * 16
        rows_0[r, pl.ds(c, 16)] = jnp.zeros((16,), jnp.float32)
        return 0
    lax.fori_loop(0, ZCHUNK * (D // 16), zbody, 0)
    def zcopy(i, _):
        pltpu.sync_copy(rows_0, acc.at[pl.ds(row0 + i * ZCHUNK, ZCHUNK)])
        return 0
    lax.fori_loop(0, NZ, zcopy, 0)
    plsc.subcore_barrier()

    bufs = (rows_0, rows_1, rows_2)
    gsems = (gsem0, gsem1, gsem2)
    ssems = (ssem0, ssem1, ssem2)

    H = CK // 2

    def start_gather(j, b):
        pltpu.async_copy(x_hbm.at[src_v.at[j, pl.ds(0, H)]],
                         bufs[b].at[pl.ds(0, H)], gsems[b])
        pltpu.async_copy(x_hbm.at[src_v.at[j, pl.ds(H, H)]],
                         bufs[b].at[pl.ds(H, H)], gsems[b])

    def wait_gather(b):
        pltpu.make_async_copy(x_hbm.at[src_v.at[0, pl.ds(0, H)]],
                              bufs[b].at[pl.ds(0, H)], gsems[b]).wait()
        pltpu.make_async_copy(x_hbm.at[src_v.at[0, pl.ds(H, H)]],
                              bufs[b].at[pl.ds(H, H)], gsems[b]).wait()

    def start_scatter(j, b):
        pltpu.async_copy(bufs[b], acc.at[dst_v.at[j]], ssems[b], add=True)

    def wait_scatter(b):
        pltpu.make_async_copy(bufs[b], acc.at[dst_v.at[0]], ssems[b]).wait()

    def group_body(g, _):
        pltpu.sync_copy(src_hbm.at[wid, g], src_v)
        pltpu.sync_copy(dst_hbm.at[wid, g], dst_v)
        start_gather(0, 0)
        start_gather(1, 1)
        wait_gather(0)
        start_scatter(0, 0)
        start_gather(2, 2)

        def edge_body(t, _):
            for i in range(3):
                j = 3 * t + 1 + i
                s = (1 + i) % 3
                wait_gather(s)
                start_scatter(j, s)
                wait_scatter((s + 2) % 3)

                @pl.when(j + 2 <= G - 1)
                def _():
                    start_gather(j + 2, (s + 2) % 3)
            return 0
        lax.fori_loop(0, (G - 1) // 3, edge_body, 0, unroll=False)
        wait_scatter((G - 1) % 3)
        return 0
    lax.fori_loop(0, NG, group_body, 0, unroll=False)

    plsc.subcore_barrier()

    def ocopy(i, _):
        pltpu.sync_copy(acc.at[pl.ds(row0 + i * ZCHUNK, ZCHUNK)],
                        out_hbm.at[cid, pl.ds(row0 + i * ZCHUNK, ZCHUNK)])
        return 0
    lax.fori_loop(0, NZ, ocopy, 0)


_sc_scatter = functools.partial(
    pl.kernel,
    out_type=jax.ShapeDtypeStruct((NC, N, D), jnp.float32),
    mesh=plsc.VectorSubcoreMesh(core_axis_name="c", subcore_axis_name="s"),
    scratch_types=[
        pltpu.VMEM_SHARED((N, D), jnp.float32),
        pltpu.VMEM((G, CK), jnp.int32),
        pltpu.VMEM((G, CK), jnp.int32),
        pltpu.VMEM((CK, D), jnp.float32),
        pltpu.VMEM((CK, D), jnp.float32),
        pltpu.VMEM((CK, D), jnp.float32),
        pltpu.SemaphoreType.DMA,
        pltpu.SemaphoreType.DMA,
        pltpu.SemaphoreType.DMA,
        pltpu.SemaphoreType.DMA,
        pltpu.SemaphoreType.DMA,
        pltpu.SemaphoreType.DMA,
    ],
)(_sc_scatter_body)


def _tc_body(p_ref, x_ref, epsb_ref, W1_ref, b1_ref, W2_ref, b2_ref,
             gamma_ref, beta_ref, o_ref):
    agg = p_ref[0] + p_ref[1] + epsb_ref[0, 0] * x_ref[...]
    h1 = jax.lax.dot_general(agg, W1_ref[...], (((1,), (1,)), ((), ())),
                             preferred_element_type=jnp.float32)
    h1 = jnp.maximum(h1 + b1_ref[...], 0.0)
    h2 = jax.lax.dot_general(h1, W2_ref[...], (((1,), (1,)), ((), ())),
                             preferred_element_type=jnp.float32)
    h2 = h2 + b2_ref[...]
    mean = jnp.mean(h2, axis=0, keepdims=True)
    var = jnp.mean(h2 * h2, axis=0, keepdims=True) - mean * mean
    inv = jax.lax.rsqrt(var + BN_EPS_CONST)
    o_ref[...] = jnp.maximum(
        (h2 - mean) * (inv * gamma_ref[...]) + beta_ref[...], 0.0)


def kernel(x, edge_index, W1, b1, W2, b2, gamma, beta, epsilon):
    src = edge_index[0].reshape(NW, NG, G, CK)
    dst = edge_index[1].reshape(NW, NG, G, CK)
    partials = _sc_scatter(x, src, dst)

    epsb = jnp.reshape(1.0 + epsilon, (1, 1)).astype(jnp.float32)
    out = pl.pallas_call(
        _tc_body,
        out_shape=jax.ShapeDtypeStruct((N, D), jnp.float32),
    )(partials, x, epsb, W1, b1.reshape(1, D), W2, b2.reshape(1, D),
      gamma.reshape(1, D), beta.reshape(1, D))
    return out


# speedup vs baseline: 11.6417x; 1.0088x over previous
import functools

import jax
import jax.numpy as jnp
from jax import lax
from jax.experimental import pallas as pl
from jax.experimental.pallas import tpu as pltpu
from jax.experimental.pallas import tpu_sc as plsc

N = 10000
D = 128
E = 320000
BN_EPS_CONST = 1e-5

NC = 2
NS = 16
NW = NC * NS
CK = 80
CHUNKS_PER_W = E // NW // CK
G = 25
NG = CHUNKS_PER_W // G

ROW_STRIDE = 624
ZCHUNK = 80
NZ = 8


def _sc_scatter_body(x_hbm, src_hbm, dst_hbm, out_hbm, acc, src_v, dst_v,
                     rows_0, rows_1, rows_2, gsem0, gsem1, gsem2,
                     ssem0, ssem1, ssem2):
    cid = lax.axis_index("c")
    sid = lax.axis_index("s")
    wid = sid * NC + cid
    row0 = sid * ROW_STRIDE

    def zbody(i, _):
        r = i // (D // 16)
        c = (i % (D // 16)) * 16
        rows_0[r, pl.ds(c, 16)] = jnp.zeros((16,), jnp.float32)
        return 0
    lax.fori_loop(0, ZCHUNK * (D // 16), zbody, 0)
    def zcopy(i, _):
        pltpu.sync_copy(rows_0, acc.at[pl.ds(row0 + i * ZCHUNK, ZCHUNK)])
        return 0
    lax.fori_loop(0, NZ, zcopy, 0)
    plsc.subcore_barrier()

    bufs = (rows_0, rows_1, rows_2)
    gsems = (gsem0, gsem1, gsem2)
    ssems = (ssem0, ssem1, ssem2)

    H = CK // 2

    def start_gather(j, b):
        pltpu.async_copy(x_hbm.at[src_v.at[j, pl.ds(0, H)]],
                         bufs[b].at[pl.ds(0, H)], gsems[b])
        pltpu.async_copy(x_hbm.at[src_v.at[j, pl.ds(H, H)]],
                         bufs[b].at[pl.ds(H, H)], gsems[b])

    def wait_gather(b):
        pltpu.make_async_copy(x_hbm.at[src_v.at[0, pl.ds(0, H)]],
                              bufs[b].at[pl.ds(0, H)], gsems[b]).wait()
        pltpu.make_async_copy(x_hbm.at[src_v.at[0, pl.ds(H, H)]],
                              bufs[b].at[pl.ds(H, H)], gsems[b]).wait()

    def start_scatter(j, b):
        pltpu.async_copy(bufs[b], acc.at[dst_v.at[j]], ssems[b], add=True)

    def wait_scatter(b):
        pltpu.make_async_copy(bufs[b], acc.at[dst_v.at[0]], ssems[b]).wait()

    def group_body(g, _):
        pltpu.sync_copy(src_hbm.at[wid, g], src_v)
        pltpu.sync_copy(dst_hbm.at[wid, g], dst_v)
        start_gather(0, 0)
        start_gather(1, 1)
        start_gather(2, 2)
        wait_gather(0)
        start_scatter(0, 0)

        def edge_body(t, _):
            for i in range(3):
                j = 3 * t + 1 + i
                s = (1 + i) % 3
                wait_scatter((s + 2) % 3)

                @pl.when(j + 2 <= G - 1)
                def _():
                    start_gather(j + 2, (s + 2) % 3)
                wait_gather(s)
                start_scatter(j, s)
            return 0
        lax.fori_loop(0, (G - 1) // 3, edge_body, 0, unroll=False)
        wait_scatter((G - 1) % 3)
        return 0
    lax.fori_loop(0, NG, group_body, 0, unroll=False)

    plsc.subcore_barrier()

    def ocopy(i, _):
        pltpu.sync_copy(acc.at[pl.ds(row0 + i * ZCHUNK, ZCHUNK)],
                        out_hbm.at[cid, pl.ds(row0 + i * ZCHUNK, ZCHUNK)])
        return 0
    lax.fori_loop(0, NZ, ocopy, 0)


_sc_scatter = functools.partial(
    pl.kernel,
    out_type=jax.ShapeDtypeStruct((NC, N, D), jnp.float32),
    mesh=plsc.VectorSubcoreMesh(core_axis_name="c", subcore_axis_name="s"),
    scratch_types=[
        pltpu.VMEM_SHARED((N, D), jnp.float32),
        pltpu.VMEM((G, CK), jnp.int32),
        pltpu.VMEM((G, CK), jnp.int32),
        pltpu.VMEM((CK, D), jnp.float32),
        pltpu.VMEM((CK, D), jnp.float32),
        pltpu.VMEM((CK, D), jnp.float32),
        pltpu.SemaphoreType.DMA,
        pltpu.SemaphoreType.DMA,
        pltpu.SemaphoreType.DMA,
        pltpu.SemaphoreType.DMA,
        pltpu.SemaphoreType.DMA,
        pltpu.SemaphoreType.DMA,
    ],
)(_sc_scatter_body)


def _tc_body(p_ref, x_ref, epsb_ref, W1_ref, b1_ref, W2_ref, b2_ref,
             gamma_ref, beta_ref, o_ref):
    agg = p_ref[0] + p_ref[1] + epsb_ref[0, 0] * x_ref[...]
    h1 = jax.lax.dot_general(agg, W1_ref[...], (((1,), (1,)), ((), ())),
                             preferred_element_type=jnp.float32)
    h1 = jnp.maximum(h1 + b1_ref[...], 0.0)
    h2 = jax.lax.dot_general(h1, W2_ref[...], (((1,), (1,)), ((), ())),
                             preferred_element_type=jnp.float32)
    h2 = h2 + b2_ref[...]
    mean = jnp.mean(h2, axis=0, keepdims=True)
    var = jnp.mean(h2 * h2, axis=0, keepdims=True) - mean * mean
    inv = jax.lax.rsqrt(var + BN_EPS_CONST)
    o_ref[...] = jnp.maximum(
        (h2 - mean) * (inv * gamma_ref[...]) + beta_ref[...], 0.0)


def kernel(x, edge_index, W1, b1, W2, b2, gamma, beta, epsilon):
    src = edge_index[0].reshape(NW, NG, G, CK)
    dst = edge_index[1].reshape(NW, NG, G, CK)
    partials = _sc_scatter(x, src, dst)

    epsb = jnp.reshape(1.0 + epsilon, (1, 1)).astype(jnp.float32)
    out = pl.pallas_call(
        _tc_body,
        out_shape=jax.ShapeDtypeStruct((N, D), jnp.float32),
    )(partials, x, epsb, W1, b1.reshape(1, D), W2, b2.reshape(1, D),
      gamma.reshape(1, D), beta.reshape(1, D))
    return out


# speedup vs baseline: 11.8386x; 1.0169x over previous
import functools

import jax
import jax.numpy as jnp
from jax import lax
from jax.experimental import pallas as pl
from jax.experimental.pallas import tpu as pltpu
from jax.experimental.pallas import tpu_sc as plsc

N = 10000
D = 128
E = 320000
BN_EPS_CONST = 1e-5

NC = 2
NS = 16
NW = NC * NS
CK = 80
CHUNKS_PER_W = E // NW // CK
G = 25
NG = CHUNKS_PER_W // G

ROW_STRIDE = 624
ZCHUNK = 80
NZ = 8


def _sc_scatter_body(x_hbm, src_hbm, dst_hbm, out_hbm, acc, src_v, dst_v,
                     rows_0, rows_1, rows_2, gsem0, gsem1, gsem2,
                     ssem0, ssem1, ssem2):
    cid = lax.axis_index("c")
    sid = lax.axis_index("s")
    wid = sid * NC + cid
    row0 = sid * ROW_STRIDE

    def zbody(i, _):
        r = i // (D // 16)
        c = (i % (D // 16)) * 16
        rows_0[r, pl.ds(c, 16)] = jnp.zeros((16,), jnp.float32)
        return 0
    lax.fori_loop(0, ZCHUNK * (D // 16), zbody, 0)
    def zcopy(i, _):
        pltpu.async_copy(rows_0, acc.at[pl.ds(row0 + i * ZCHUNK, ZCHUNK)],
                         ssem0)
        return 0
    lax.fori_loop(0, NZ, zcopy, 0)
    def zwait(i, _):
        pltpu.make_async_copy(rows_0, acc.at[pl.ds(row0, ZCHUNK)],
                              ssem0).wait()
        return 0
    lax.fori_loop(0, NZ, zwait, 0)
    plsc.subcore_barrier()

    bufs = (rows_0, rows_1, rows_2)
    gsems = (gsem0, gsem1, gsem2)
    ssems = (ssem0, ssem1, ssem2)

    H = CK // 2

    def start_gather(j, b):
        pltpu.async_copy(x_hbm.at[src_v.at[j, pl.ds(0, H)]],
                         bufs[b].at[pl.ds(0, H)], gsems[b])
        pltpu.async_copy(x_hbm.at[src_v.at[j, pl.ds(H, H)]],
                         bufs[b].at[pl.ds(H, H)], gsems[b])

    def wait_gather(b):
        pltpu.make_async_copy(x_hbm.at[src_v.at[0, pl.ds(0, H)]],
                              bufs[b].at[pl.ds(0, H)], gsems[b]).wait()
        pltpu.make_async_copy(x_hbm.at[src_v.at[0, pl.ds(H, H)]],
                              bufs[b].at[pl.ds(H, H)], gsems[b]).wait()

    def start_scatter(j, b):
        pltpu.async_copy(bufs[b], acc.at[dst_v.at[j]], ssems[b], add=True)

    def wait_scatter(b):
        pltpu.make_async_copy(bufs[b], acc.at[dst_v.at[0]], ssems[b]).wait()

    def group_body(g, _):
        pltpu.async_copy(src_hbm.at[wid, g], src_v, gsem0)
        pltpu.async_copy(dst_hbm.at[wid, g], dst_v, gsem0)
        pltpu.make_async_copy(src_hbm.at[wid, g], src_v, gsem0).wait()
        pltpu.make_async_copy(dst_hbm.at[wid, g], dst_v, gsem0).wait()
        start_gather(0, 0)
        start_gather(1, 1)
        start_gather(2, 2)
        wait_gather(0)
        start_scatter(0, 0)

        def edge_body(t, _):
            for i in range(3):
                j = 3 * t + 1 + i
                s = (1 + i) % 3
                wait_scatter((s + 2) % 3)

                @pl.when(j + 2 <= G - 1)
                def _():
                    start_gather(j + 2, (s + 2) % 3)
                wait_gather(s)
                start_scatter(j, s)
            return 0
        lax.fori_loop(0, (G - 1) // 3, edge_body, 0, unroll=False)
        wait_scatter((G - 1) % 3)
        return 0
    lax.fori_loop(0, NG, group_body, 0, unroll=False)

    plsc.subcore_barrier()

    def ocopy(i, _):
        pltpu.async_copy(acc.at[pl.ds(row0 + i * ZCHUNK, ZCHUNK)],
                         out_hbm.at[cid, pl.ds(row0 + i * ZCHUNK, ZCHUNK)],
                         ssem0)
        return 0
    lax.fori_loop(0, NZ, ocopy, 0)
    def owait(i, _):
        pltpu.make_async_copy(acc.at[pl.ds(row0, ZCHUNK)],
                              out_hbm.at[cid, pl.ds(row0, ZCHUNK)],
                              ssem0).wait()
        return 0
    lax.fori_loop(0, NZ, owait, 0)


_sc_scatter = functools.partial(
    pl.kernel,
    out_type=jax.ShapeDtypeStruct((NC, N, D), jnp.float32),
    mesh=plsc.VectorSubcoreMesh(core_axis_name="c", subcore_axis_name="s"),
    scratch_types=[
        pltpu.VMEM_SHARED((N, D), jnp.float32),
        pltpu.VMEM((G, CK), jnp.int32),
        pltpu.VMEM((G, CK), jnp.int32),
        pltpu.VMEM((CK, D), jnp.float32),
        pltpu.VMEM((CK, D), jnp.float32),
        pltpu.VMEM((CK, D), jnp.float32),
        pltpu.SemaphoreType.DMA,
        pltpu.SemaphoreType.DMA,
        pltpu.SemaphoreType.DMA,
        pltpu.SemaphoreType.DMA,
        pltpu.SemaphoreType.DMA,
        pltpu.SemaphoreType.DMA,
    ],
)(_sc_scatter_body)


def _tc_body(p_ref, x_ref, epsb_ref, W1_ref, b1_ref, W2_ref, b2_ref,
             gamma_ref, beta_ref, o_ref):
    agg = p_ref[0] + p_ref[1] + epsb_ref[0, 0] * x_ref[...]
    h1 = jax.lax.dot_general(agg, W1_ref[...], (((1,), (1,)), ((), ())),
                             preferred_element_type=jnp.float32)
    h1 = jnp.maximum(h1 + b1_ref[...], 0.0)
    h2 = jax.lax.dot_general(h1, W2_ref[...], (((1,), (1,)), ((), ())),
                             preferred_element_type=jnp.float32)
    h2 = h2 + b2_ref[...]
    mean = jnp.mean(h2, axis=0, keepdims=True)
    var = jnp.mean(h2 * h2, axis=0, keepdims=True) - mean * mean
    inv = jax.lax.rsqrt(var + BN_EPS_CONST)
    o_ref[...] = jnp.maximum(
        (h2 - mean) * (inv * gamma_ref[...]) + beta_ref[...], 0.0)


def kernel(x, edge_index, W1, b1, W2, b2, gamma, beta, epsilon):
    src = edge_index[0].reshape(NW, NG, G, CK)
    dst = edge_index[1].reshape(NW, NG, G, CK)
    partials = _sc_scatter(x, src, dst)

    epsb = jnp.reshape(1.0 + epsilon, (1, 1)).astype(jnp.float32)
    out = pl.pallas_call(
        _tc_body,
        out_shape=jax.ShapeDtypeStruct((N, D), jnp.float32),
    )(partials, x, epsb, W1, b1.reshape(1, D), W2, b2.reshape(1, D),
      gamma.reshape(1, D), beta.reshape(1, D))
    return out


# speedup vs baseline: 12.8399x; 1.0846x over previous
import functools

import jax
import jax.numpy as jnp
from jax import lax
from jax.experimental import pallas as pl
from jax.experimental.pallas import tpu as pltpu
from jax.experimental.pallas import tpu_sc as plsc

N = 10000
D = 128
E = 320000
BN_EPS_CONST = 1e-5

NC = 2
NS = 16
NW = NC * NS
CK = 80
CHUNKS_PER_W = E // NW // CK
G = 25
NG = CHUNKS_PER_W // G

ROW_STRIDE = 624
ZCHUNK = 80
NZ = 8


def _sc_scatter_body(x_hbm, ei_hbm, out_hbm, acc, src_v, dst_v,
                     rows_0, rows_1, rows_2, gsem0, gsem1, gsem2,
                     ssem0, ssem1, ssem2):
    cid = lax.axis_index("c")
    sid = lax.axis_index("s")
    wid = sid * NC + cid
    row0 = sid * ROW_STRIDE

    def zbody(i, _):
        r = i // (D // 16)
        c = (i % (D // 16)) * 16
        rows_0[r, pl.ds(c, 16)] = jnp.zeros((16,), jnp.float32)
        return 0
    lax.fori_loop(0, ZCHUNK * (D // 16), zbody, 0)
    def zcopy(i, _):
        pltpu.async_copy(rows_0, acc.at[pl.ds(row0 + i * ZCHUNK, ZCHUNK)],
                         ssem0)
        return 0
    lax.fori_loop(0, NZ, zcopy, 0)
    def zwait(i, _):
        pltpu.make_async_copy(rows_0, acc.at[pl.ds(row0, ZCHUNK)],
                              ssem0).wait()
        return 0
    lax.fori_loop(0, NZ, zwait, 0)
    plsc.subcore_barrier()

    bufs = (rows_0, rows_1, rows_2)
    gsems = (gsem0, gsem1, gsem2)
    ssems = (ssem0, ssem1, ssem2)

    H = CK // 2

    def start_gather(j, b):
        pltpu.async_copy(x_hbm.at[src_v.at[j, pl.ds(0, H)]],
                         bufs[b].at[pl.ds(0, H)], gsems[b])
        pltpu.async_copy(x_hbm.at[src_v.at[j, pl.ds(H, H)]],
                         bufs[b].at[pl.ds(H, H)], gsems[b])

    def wait_gather(b):
        pltpu.make_async_copy(x_hbm.at[src_v.at[0, pl.ds(0, H)]],
                              bufs[b].at[pl.ds(0, H)], gsems[b]).wait()
        pltpu.make_async_copy(x_hbm.at[src_v.at[0, pl.ds(H, H)]],
                              bufs[b].at[pl.ds(H, H)], gsems[b]).wait()

    def start_scatter(j, b):
        pltpu.async_copy(bufs[b], acc.at[dst_v.at[j]], ssems[b], add=True)

    def wait_scatter(b):
        pltpu.make_async_copy(bufs[b], acc.at[dst_v.at[0]], ssems[b]).wait()

    def group_body(g, _):
        pltpu.async_copy(ei_hbm.at[0, wid, g], src_v, gsem0)
        pltpu.async_copy(ei_hbm.at[1, wid, g], dst_v, gsem0)
        pltpu.make_async_copy(ei_hbm.at[0, wid, g], src_v, gsem0).wait()
        pltpu.make_async_copy(ei_hbm.at[1, wid, g], dst_v, gsem0).wait()
        start_gather(0, 0)
        start_gather(1, 1)
        start_gather(2, 2)
        wait_gather(0)
        start_scatter(0, 0)

        def edge_body(t, _):
            for i in range(3):
                j = 3 * t + 1 + i
                s = (1 + i) % 3
                wait_scatter((s + 2) % 3)

                @pl.when(j + 2 <= G - 1)
                def _():
                    start_gather(j + 2, (s + 2) % 3)
                wait_gather(s)
                start_scatter(j, s)
            return 0
        lax.fori_loop(0, (G - 1) // 3, edge_body, 0, unroll=False)
        wait_scatter((G - 1) % 3)
        return 0
    lax.fori_loop(0, NG, group_body, 0, unroll=False)

    plsc.subcore_barrier()

    def ocopy(i, _):
        pltpu.async_copy(acc.at[pl.ds(row0 + i * ZCHUNK, ZCHUNK)],
                         out_hbm.at[cid, pl.ds(row0 + i * ZCHUNK, ZCHUNK)],
                         ssem0)
        return 0
    lax.fori_loop(0, NZ, ocopy, 0)
    def owait(i, _):
        pltpu.make_async_copy(acc.at[pl.ds(row0, ZCHUNK)],
                              out_hbm.at[cid, pl.ds(row0, ZCHUNK)],
                              ssem0).wait()
        return 0
    lax.fori_loop(0, NZ, owait, 0)


_sc_scatter = functools.partial(
    pl.kernel,
    out_type=jax.ShapeDtypeStruct((NC, N, D), jnp.float32),
    mesh=plsc.VectorSubcoreMesh(core_axis_name="c", subcore_axis_name="s"),
    scratch_types=[
        pltpu.VMEM_SHARED((N, D), jnp.float32),
        pltpu.VMEM((G, CK), jnp.int32),
        pltpu.VMEM((G, CK), jnp.int32),
        pltpu.VMEM((CK, D), jnp.float32),
        pltpu.VMEM((CK, D), jnp.float32),
        pltpu.VMEM((CK, D), jnp.float32),
        pltpu.SemaphoreType.DMA,
        pltpu.SemaphoreType.DMA,
        pltpu.SemaphoreType.DMA,
        pltpu.SemaphoreType.DMA,
        pltpu.SemaphoreType.DMA,
        pltpu.SemaphoreType.DMA,
    ],
)(_sc_scatter_body)


def _tc_body(p_ref, x_ref, epsb_ref, W1_ref, b1_ref, W2_ref, b2_ref,
             gamma_ref, beta_ref, o_ref):
    agg = p_ref[0] + p_ref[1] + epsb_ref[0, 0] * x_ref[...]
    h1 = jax.lax.dot_general(agg, W1_ref[...], (((1,), (1,)), ((), ())),
                             preferred_element_type=jnp.float32)
    h1 = jnp.maximum(h1 + b1_ref[...], 0.0)
    h2 = jax.lax.dot_general(h1, W2_ref[...], (((1,), (1,)), ((), ())),
                             preferred_element_type=jnp.float32)
    h2 = h2 + b2_ref[...]
    mean = jnp.mean(h2, axis=0, keepdims=True)
    var = jnp.mean(h2 * h2, axis=0, keepdims=True) - mean * mean
    inv = jax.lax.rsqrt(var + BN_EPS_CONST)
    o_ref[...] = jnp.maximum(
        (h2 - mean) * (inv * gamma_ref[...]) + beta_ref[...], 0.0)


def kernel(x, edge_index, W1, b1, W2, b2, gamma, beta, epsilon):
    ei = edge_index.reshape(2, NW, NG, G, CK)
    partials = _sc_scatter(x, ei)

    epsb = jnp.reshape(1.0 + epsilon, (1, 1)).astype(jnp.float32)
    out = pl.pallas_call(
        _tc_body,
        out_shape=jax.ShapeDtypeStruct((N, D), jnp.float32),
    )(partials, x, epsb, W1, b1.reshape(1, D), W2, b2.reshape(1, D),
      gamma.reshape(1, D), beta.reshape(1, D))
    return out


# speedup vs baseline: 12.8689x; 1.0023x over previous
import functools

import jax
import jax.numpy as jnp
from jax import lax
from jax.experimental import pallas as pl
from jax.experimental.pallas import tpu as pltpu
from jax.experimental.pallas import tpu_sc as plsc

N = 10000
D = 128
E = 320000
BN_EPS_CONST = 1e-5

NC = 2
NS = 16
NW = NC * NS
CK = 80
CHUNKS_PER_W = E // NW // CK
G = 25
NG = CHUNKS_PER_W // G

ROW_STRIDE = 624
ZCHUNK = 80
NZ = 8


def _sc_scatter_body(x_hbm, ei_hbm, out_hbm, acc, src_v, dst_v,
                     rows_0, rows_1, rows_2, gsem0, gsem1, gsem2,
                     ssem0, ssem1, ssem2):
    cid = lax.axis_index("c")
    sid = lax.axis_index("s")
    wid = sid * NC + cid
    row0 = sid * ROW_STRIDE

    bufs = (rows_0, rows_1, rows_2)
    gsems = (gsem0, gsem1, gsem2)
    ssems = (ssem0, ssem1, ssem2)

    H = CK // 2

    def start_gather(j, b):
        pltpu.async_copy(x_hbm.at[src_v.at[j, pl.ds(0, H)]],
                         bufs[b].at[pl.ds(0, H)], gsems[b])
        pltpu.async_copy(x_hbm.at[src_v.at[j, pl.ds(H, H)]],
                         bufs[b].at[pl.ds(H, H)], gsems[b])

    def wait_gather(b):
        pltpu.make_async_copy(x_hbm.at[src_v.at[0, pl.ds(0, H)]],
                              bufs[b].at[pl.ds(0, H)], gsems[b]).wait()
        pltpu.make_async_copy(x_hbm.at[src_v.at[0, pl.ds(H, H)]],
                              bufs[b].at[pl.ds(H, H)], gsems[b]).wait()

    def start_scatter(j, b):
        pltpu.async_copy(bufs[b], acc.at[dst_v.at[j]], ssems[b], add=True)

    def wait_scatter(b):
        pltpu.make_async_copy(bufs[b], acc.at[dst_v.at[0]], ssems[b]).wait()

    def load_idx(g):
        pltpu.async_copy(ei_hbm.at[0, wid, g], src_v, gsem0)
        pltpu.async_copy(ei_hbm.at[1, wid, g], dst_v, gsem0)
        pltpu.make_async_copy(ei_hbm.at[0, wid, g], src_v, gsem0).wait()
        pltpu.make_async_copy(ei_hbm.at[1, wid, g], dst_v, gsem0).wait()

    def group_tail(first):
        wait_gather(0)
        start_scatter(0, 0)

        def edge_body(t, _):
            for i in range(3):
                j = 3 * t + 1 + i
                s = (1 + i) % 3
                wait_scatter((s + 2) % 3)

                @pl.when(j + 2 <= G - 1)
                def _():
                    start_gather(j + 2, (s + 2) % 3)
                wait_gather(s)
                start_scatter(j, s)
            return 0
        lax.fori_loop(0, (G - 1) // 3, edge_body, 0, unroll=False)
        wait_scatter((G - 1) % 3)

    load_idx(0)
    def zbody(i, _):
        r = i // (D // 16)
        c = (i % (D // 16)) * 16
        rows_0[r, pl.ds(c, 16)] = jnp.zeros((16,), jnp.float32)
        return 0
    lax.fori_loop(0, ZCHUNK * (D // 16), zbody, 0)
    def zcopy(i, _):
        pltpu.async_copy(rows_0, acc.at[pl.ds(row0 + i * ZCHUNK, ZCHUNK)],
                         ssem0)
        return 0
    lax.fori_loop(0, NZ, zcopy, 0)
    start_gather(1, 1)
    start_gather(2, 2)
    def zwait(i, _):
        pltpu.make_async_copy(rows_0, acc.at[pl.ds(row0, ZCHUNK)],
                              ssem0).wait()
        return 0
    lax.fori_loop(0, NZ, zwait, 0)
    plsc.subcore_barrier()
    start_gather(0, 0)
    group_tail(True)

    def group_body(g, _):
        load_idx(g)
        start_gather(0, 0)
        start_gather(1, 1)
        start_gather(2, 2)
        group_tail(False)
        return 0
    lax.fori_loop(1, NG, group_body, 0, unroll=False)

    plsc.subcore_barrier()

    def ocopy(i, _):
        pltpu.async_copy(acc.at[pl.ds(row0 + i * ZCHUNK, ZCHUNK)],
                         out_hbm.at[cid, pl.ds(row0 + i * ZCHUNK, ZCHUNK)],
                         ssem0)
        return 0
    lax.fori_loop(0, NZ, ocopy, 0)
    def owait(i, _):
        pltpu.make_async_copy(acc.at[pl.ds(row0, ZCHUNK)],
                              out_hbm.at[cid, pl.ds(row0, ZCHUNK)],
                              ssem0).wait()
        return 0
    lax.fori_loop(0, NZ, owait, 0)


_sc_scatter = functools.partial(
    pl.kernel,
    out_type=jax.ShapeDtypeStruct((NC, N, D), jnp.float32),
    mesh=plsc.VectorSubcoreMesh(core_axis_name="c", subcore_axis_name="s"),
    scratch_types=[
        pltpu.VMEM_SHARED((N, D), jnp.float32),
        pltpu.VMEM((G, CK), jnp.int32),
        pltpu.VMEM((G, CK), jnp.int32),
        pltpu.VMEM((CK, D), jnp.float32),
        pltpu.VMEM((CK, D), jnp.float32),
        pltpu.VMEM((CK, D), jnp.float32),
        pltpu.SemaphoreType.DMA,
        pltpu.SemaphoreType.DMA,
        pltpu.SemaphoreType.DMA,
        pltpu.SemaphoreType.DMA,
        pltpu.SemaphoreType.DMA,
        pltpu.SemaphoreType.DMA,
    ],
)(_sc_scatter_body)


def _tc_body(p_ref, x_ref, epsb_ref, W1_ref, b1_ref, W2_ref, b2_ref,
             gamma_ref, beta_ref, o_ref):
    agg = p_ref[0] + p_ref[1] + epsb_ref[0, 0] * x_ref[...]
    h1 = jax.lax.dot_general(agg, W1_ref[...], (((1,), (1,)), ((), ())),
                             preferred_element_type=jnp.float32)
    h1 = jnp.maximum(h1 + b1_ref[...], 0.0)
    h2 = jax.lax.dot_general(h1, W2_ref[...], (((1,), (1,)), ((), ())),
                             preferred_element_type=jnp.float32)
    h2 = h2 + b2_ref[...]
    mean = jnp.mean(h2, axis=0, keepdims=True)
    var = jnp.mean(h2 * h2, axis=0, keepdims=True) - mean * mean
    inv = jax.lax.rsqrt(var + BN_EPS_CONST)
    o_ref[...] = jnp.maximum(
        (h2 - mean) * (inv * gamma_ref[...]) + beta_ref[...], 0.0)


def kernel(x, edge_index, W1, b1, W2, b2, gamma, beta, epsilon):
    ei = edge_index.reshape(2, NW, NG, G, CK)
    partials = _sc_scatter(x, ei)

    epsb = jnp.reshape(1.0 + epsilon, (1, 1)).astype(jnp.float32)
    out = pl.pallas_call(
        _tc_body,
        out_shape=jax.ShapeDtypeStruct((N, D), jnp.float32),
    )(partials, x, epsb, W1, b1.reshape(1, D), W2, b2.reshape(1, D),
      gamma.reshape(1, D), beta.reshape(1, D))
    return out


# speedup vs baseline: 13.2747x; 1.0315x over previous
import functools

import jax
import jax.numpy as jnp
from jax import lax
from jax.experimental import pallas as pl
from jax.experimental.pallas import tpu as pltpu
from jax.experimental.pallas import tpu_sc as plsc

N = 10000
D = 128
E = 320000
BN_EPS_CONST = 1e-5

NC = 2
NS = 16
NW = NC * NS
CK = 80
CHUNKS_PER_W = E // NW // CK
G = 25
NG = CHUNKS_PER_W // G

ROW_STRIDE = 624
ZCHUNK = 80
NZ = 8


def _sc_scatter_body(x_hbm, ei_hbm, out_hbm, acc, src_v, dst_v,
                     src_w, dst_w,
                     rows_0, rows_1, rows_2, gsem0, gsem1, gsem2,
                     ssem0, ssem1, ssem2, isem):
    cid = lax.axis_index("c")
    sid = lax.axis_index("s")
    wid = sid * NC + cid
    row0 = sid * ROW_STRIDE

    bufs = (rows_0, rows_1, rows_2)
    gsems = (gsem0, gsem1, gsem2)
    ssems = (ssem0, ssem1, ssem2)
    idx_ab = ((src_v, dst_v), (src_w, dst_w))

    H = CK // 2

    def start_gather(sv, j, b):
        pltpu.async_copy(x_hbm.at[sv.at[j, pl.ds(0, H)]],
                         bufs[b].at[pl.ds(0, H)], gsems[b])
        pltpu.async_copy(x_hbm.at[sv.at[j, pl.ds(H, H)]],
                         bufs[b].at[pl.ds(H, H)], gsems[b])

    def wait_gather(b):
        pltpu.make_async_copy(x_hbm.at[src_v.at[0, pl.ds(0, H)]],
                              bufs[b].at[pl.ds(0, H)], gsems[b]).wait()
        pltpu.make_async_copy(x_hbm.at[src_v.at[0, pl.ds(H, H)]],
                              bufs[b].at[pl.ds(H, H)], gsems[b]).wait()

    def start_scatter(dv, j, b):
        pltpu.async_copy(bufs[b], acc.at[dv.at[j]], ssems[b], add=True)

    def wait_scatter(b):
        pltpu.make_async_copy(bufs[b], acc.at[dst_v.at[0]], ssems[b]).wait()

    def load_idx(g, sv, dv):
        pltpu.async_copy(ei_hbm.at[0, wid, g], sv, isem)
        pltpu.async_copy(ei_hbm.at[1, wid, g], dv, isem)

    def wait_idx(sv, dv):
        pltpu.make_async_copy(ei_hbm.at[0, wid, 0], sv, isem).wait()
        pltpu.make_async_copy(ei_hbm.at[1, wid, 0], dv, isem).wait()

    def edge_loop(sv, dv):
        wait_gather(0)
        start_scatter(dv, 0, 0)

        def edge_body(t, _):
            for i in range(3):
                j = 3 * t + 1 + i
                s = (1 + i) % 3
                wait_scatter((s + 2) % 3)

                @pl.when(j + 2 <= G - 1)
                def _():
                    start_gather(sv, j + 2, (s + 2) % 3)
                wait_gather(s)
                start_scatter(dv, j, s)
            return 0
        lax.fori_loop(0, (G - 1) // 3, edge_body, 0, unroll=False)

    sv, dv = idx_ab[0]
    load_idx(0, sv, dv)

    def zbody(i, _):
        r = i // (D // 16)
        c = (i % (D // 16)) * 16
        rows_0[r, pl.ds(c, 16)] = jnp.zeros((16,), jnp.float32)
        return 0
    lax.fori_loop(0, ZCHUNK * (D // 16), zbody, 0)

    def zcopy(i, _):
        pltpu.async_copy(rows_0, acc.at[pl.ds(row0 + i * ZCHUNK, ZCHUNK)],
                         ssem0)
        return 0
    lax.fori_loop(0, NZ, zcopy, 0)
    wait_idx(sv, dv)
    load_idx(1, *idx_ab[1])
    start_gather(sv, 1, 1)
    start_gather(sv, 2, 2)

    def zwait(i, _):
        pltpu.make_async_copy(rows_0, acc.at[pl.ds(row0, ZCHUNK)],
                              ssem0).wait()
        return 0
    lax.fori_loop(0, NZ, zwait, 0)
    plsc.subcore_barrier()
    start_gather(sv, 0, 0)
    edge_loop(sv, dv)

    for g in range(1, NG):
        sv, dv = idx_ab[g % 2]
        wait_idx(sv, dv)
        start_gather(sv, 1, 1)
        start_gather(sv, 2, 2)
        wait_scatter(0)
        if g + 1 < NG:
            load_idx(g + 1, *idx_ab[(g + 1) % 2])
        start_gather(sv, 0, 0)
        edge_loop(sv, dv)

    wait_scatter(0)
    plsc.subcore_barrier()

    def ocopy(i, _):
        pltpu.async_copy(acc.at[pl.ds(row0 + i * ZCHUNK, ZCHUNK)],
                         out_hbm.at[cid, pl.ds(row0 + i * ZCHUNK, ZCHUNK)],
                         ssem0)
        return 0
    lax.fori_loop(0, NZ, ocopy, 0)
    def owait(i, _):
        pltpu.make_async_copy(acc.at[pl.ds(row0, ZCHUNK)],
                              out_hbm.at[cid, pl.ds(row0, ZCHUNK)],
                              ssem0).wait()
        return 0
    lax.fori_loop(0, NZ, owait, 0)


_sc_scatter = functools.partial(
    pl.kernel,
    out_type=jax.ShapeDtypeStruct((NC, N, D), jnp.float32),
    mesh=plsc.VectorSubcoreMesh(core_axis_name="c", subcore_axis_name="s"),
    scratch_types=[
        pltpu.VMEM_SHARED((N, D), jnp.float32),
        pltpu.VMEM((G, CK), jnp.int32),
        pltpu.VMEM((G, CK), jnp.int32),
        pltpu.VMEM((G, CK), jnp.int32),
        pltpu.VMEM((G, CK), jnp.int32),
        pltpu.VMEM((CK, D), jnp.float32),
        pltpu.VMEM((CK, D), jnp.float32),
        pltpu.VMEM((CK, D), jnp.float32),
        pltpu.SemaphoreType.DMA,
        pltpu.SemaphoreType.DMA,
        pltpu.SemaphoreType.DMA,
        pltpu.SemaphoreType.DMA,
        pltpu.SemaphoreType.DMA,
        pltpu.SemaphoreType.DMA,
        pltpu.SemaphoreType.DMA,
    ],
)(_sc_scatter_body)


def _tc_body(p_ref, x_ref, epsb_ref, W1_ref, b1_ref, W2_ref, b2_ref,
             gamma_ref, beta_ref, o_ref):
    agg = p_ref[0] + p_ref[1] + epsb_ref[0, 0] * x_ref[...]
    h1 = jax.lax.dot_general(agg, W1_ref[...], (((1,), (1,)), ((), ())),
                             preferred_element_type=jnp.float32)
    h1 = jnp.maximum(h1 + b1_ref[...], 0.0)
    h2 = jax.lax.dot_general(h1, W2_ref[...], (((1,), (1,)), ((), ())),
                             preferred_element_type=jnp.float32)
    h2 = h2 + b2_ref[...]
    mean = jnp.mean(h2, axis=0, keepdims=True)
    var = jnp.mean(h2 * h2, axis=0, keepdims=True) - mean * mean
    inv = jax.lax.rsqrt(var + BN_EPS_CONST)
    o_ref[...] = jnp.maximum(
        (h2 - mean) * (inv * gamma_ref[...]) + beta_ref[...], 0.0)


def kernel(x, edge_index, W1, b1, W2, b2, gamma, beta, epsilon):
    ei = edge_index.reshape(2, NW, NG, G, CK)
    partials = _sc_scatter(x, ei)

    epsb = jnp.reshape(1.0 + epsilon, (1, 1)).astype(jnp.float32)
    out = pl.pallas_call(
        _tc_body,
        out_shape=jax.ShapeDtypeStruct((N, D), jnp.float32),
    )(partials, x, epsb, W1, b1.reshape(1, D), W2, b2.reshape(1, D),
      gamma.reshape(1, D), beta.reshape(1, D))
    return out
